# Initial kernel scaffold; baseline (speedup 1.0000x reference)
import jax
import jax.numpy as jnp
from jax.experimental import pallas as pl


def kernel(x):
    raise NotImplementedError("write your pallas kernel here")


# speedup vs baseline: 116.3086x; 116.3086x over previous
import functools

import jax
import jax.numpy as jnp
from jax import lax
from jax.experimental import pallas as pl
from jax.experimental.pallas import tpu as pltpu
from jax.experimental.pallas import tpu_sc as plsc

NUM_BINS = 256
LANES = 16
CHUNK = 32768
UNROLL = 4


def _make_kernel(rows, cols):
    n_chunks = cols // CHUNK
    mesh = plsc.VectorSubcoreMesh(core_axis_name="c", subcore_axis_name="s")

    @functools.partial(
        pl.kernel,
        out_type=jax.ShapeDtypeStruct((rows, NUM_BINS), jnp.float32),
        mesh=mesh,
        scratch_types=[
            pltpu.VMEM((CHUNK,), jnp.float32),
            pltpu.VMEM((CHUNK,), jnp.float32),
            pltpu.VMEM((LANES, NUM_BINS), jnp.float32),
            pltpu.VMEM((NUM_BINS,), jnp.float32),
            pltpu.SemaphoreType.DMA,
            pltpu.SemaphoreType.DMA,
        ],
        compiler_params=pltpu.CompilerParams(needs_layout_passes=False),
    )
    def hist_kernel(x_hbm, out_hbm, buf0, buf1, acc, row_buf, sem0, sem1):
        row = lax.axis_index("s") * mesh.num_cores + lax.axis_index("c")

        zeros = jnp.zeros((LANES,), jnp.float32)
        for l in range(LANES):
            for c in range(NUM_BINS // LANES):
                acc[l, pl.ds(c * LANES, LANES)] = zeros

        lanes = lax.iota(jnp.int32, LANES)
        bufs = [buf0, buf1]
        sems = [sem0, sem1]
        descs = [None, None]
        descs[0] = pltpu.async_copy(x_hbm.at[row, pl.ds(0, CHUNK)], buf0, sem0)

        def body(i, buf):
            base = i * (LANES * UNROLL)
            for j in range(UNROLL):
                v = buf[pl.ds(base + j * LANES, LANES)]
                ib = v.astype(jnp.int32)
                fb = ib.astype(jnp.float32)
                w1 = v - fb
                w0 = 1.0 - w1
                plsc.addupdate_scatter(acc, [lanes, ib], w0)
                plsc.addupdate_scatter(acc, [lanes, ib + 1], w1)

        for k in range(n_chunks):
            nxt = (k + 1) % 2
            if k + 1 < n_chunks:
                descs[nxt] = pltpu.async_copy(
                    x_hbm.at[row, pl.ds((k + 1) * CHUNK, CHUNK)],
                    bufs[nxt], sems[nxt])
            descs[k % 2].wait()
            buf = bufs[k % 2]
            lax.fori_loop(0, CHUNK // (LANES * UNROLL),
                          lambda i, _, buf=buf: (body(i, buf), 0)[1], 0)

        for c in range(NUM_BINS // LANES):
            s = acc[0, pl.ds(c * LANES, LANES)]
            for l in range(1, LANES):
                s = s + acc[l, pl.ds(c * LANES, LANES)]
            row_buf[pl.ds(c * LANES, LANES)] = s

        pltpu.sync_copy(row_buf, out_hbm.at[row])

    return hist_kernel


@jax.jit
def kernel(x):
    rows, cols = x.shape
    return _make_kernel(rows, cols)(x)


# speedup vs baseline: 1304.0871x; 11.2123x over previous
import functools

import jax
import jax.numpy as jnp
from jax import lax
from jax.experimental import pallas as pl
from jax.experimental.pallas import tpu as pltpu
from jax.experimental.pallas import tpu_sc as plsc

NUM_BINS = 256
LANES = 16
CHUNK = 32768
UNROLL = 4


def _make_kernel(rows, cols):
    n_chunks = cols // CHUNK
    mesh = plsc.VectorSubcoreMesh(core_axis_name="c", subcore_axis_name="s")

    @functools.partial(
        pl.kernel,
        out_type=jax.ShapeDtypeStruct((rows, NUM_BINS), jnp.float32),
        mesh=mesh,
        scratch_types=[
            pltpu.VMEM((CHUNK,), jnp.float32),
            pltpu.VMEM((CHUNK,), jnp.float32),
            pltpu.VMEM((LANES, NUM_BINS), jnp.float32),
            pltpu.VMEM((NUM_BINS,), jnp.float32),
            pltpu.SemaphoreType.DMA,
            pltpu.SemaphoreType.DMA,
        ],
        compiler_params=pltpu.CompilerParams(needs_layout_passes=False),
    )
    def hist_kernel(x_hbm, out_hbm, buf0, buf1, acc, row_buf, sem0, sem1):
        row = lax.axis_index("s") * mesh.num_cores + lax.axis_index("c")

        zeros = jnp.zeros((LANES,), jnp.float32)
        for l in range(LANES):
            for c in range(NUM_BINS // LANES):
                acc[l, pl.ds(c * LANES, LANES)] = zeros

        lanes = lax.iota(jnp.int32, LANES)
        bufs = [buf0, buf1]
        sems = [sem0, sem1]
        descs = [None, None]
        descs[0] = pltpu.async_copy(x_hbm.at[row, pl.ds(0, CHUNK)], buf0, sem0)

        def process_chunk(buf):
            @functools.partial(plsc.parallel_loop, 0, CHUNK // LANES,
                               unroll=UNROLL)
            def _(i):
                v = buf[pl.ds(i * LANES, LANES)]
                ib = v.astype(jnp.int32)
                fb = ib.astype(jnp.float32)
                w1 = v - fb
                w0 = 1.0 - w1
                plsc.addupdate_scatter(acc, [lanes, ib], w0)
                plsc.addupdate_scatter(acc, [lanes, ib + 1], w1)

        for k in range(n_chunks):
            nxt = (k + 1) % 2
            if k + 1 < n_chunks:
                descs[nxt] = pltpu.async_copy(
                    x_hbm.at[row, pl.ds((k + 1) * CHUNK, CHUNK)],
                    bufs[nxt], sems[nxt])
            descs[k % 2].wait()
            process_chunk(bufs[k % 2])

        for c in range(NUM_BINS // LANES):
            s = acc[0, pl.ds(c * LANES, LANES)]
            for l in range(1, LANES):
                s = s + acc[l, pl.ds(c * LANES, LANES)]
            row_buf[pl.ds(c * LANES, LANES)] = s

        pltpu.sync_copy(row_buf, out_hbm.at[row])

    return hist_kernel


@jax.jit
def kernel(x):
    rows, cols = x.shape
    return _make_kernel(rows, cols)(x)
